# baseline (speedup 1.0000x reference)
import functools

import jax
import jax.numpy as jnp
from jax import lax
from jax.experimental import pallas as pl
from jax.experimental.pallas import tpu as pltpu
from jax.experimental.pallas import tpu_sc as plsc

N = 10000
NP = 10240
RPT = NP // 16
D = 256
DH = 128
E = 160000
CHUNK = 128
NCHUNK = 79
EP = 16 * NCHUNK * CHUNK


def _make_sc_scatter(with_cnt: bool):
    mesh = plsc.VectorSubcoreMesh(core_axis_name="c", subcore_axis_name="s")
    out_type = [
        jax.ShapeDtypeStruct((NP, DH), jnp.float32),
        jax.ShapeDtypeStruct((NP, DH), jnp.float32),
        jax.ShapeDtypeStruct((NP,), jnp.float32),
    ]
    scratch_types = [
        pltpu.VMEM((NCHUNK, CHUNK), jnp.int32),
        pltpu.VMEM((NCHUNK, CHUNK), jnp.int32),
        pltpu.VMEM((CHUNK, DH), jnp.float32),
        pltpu.VMEM((CHUNK,), jnp.float32),
        pltpu.VMEM_SHARED((NP, DH), jnp.float32),
        pltpu.VMEM_SHARED((NP,), jnp.float32),
        pltpu.SemaphoreType.DMA,
        pltpu.SemaphoreType.DMA,
    ]

    @functools.partial(pl.kernel, mesh=mesh, out_type=out_type,
                       scratch_types=scratch_types)
    def k(xa_h, xb_h, src_h, dst_h, zr_h, zc_h, ones_h,
          outa_h, outb_h, cnt_h,
          srcv, dstv, rows, onesv, acc, cacc, sem, isem):
        cid = lax.axis_index("c")
        sid = lax.axis_index("s")
        rslice = pl.ds(sid * RPT, RPT)

        pltpu.sync_copy(src_h.at[sid], srcv)
        pltpu.sync_copy(dst_h.at[sid], dstv)
        pltpu.sync_copy(zr_h, acc.at[rslice])
        if with_cnt:
            @pl.when(cid == 0)
            def _():
                pltpu.sync_copy(ones_h, onesv)
                pltpu.sync_copy(zc_h, cacc.at[rslice])
        plsc.subcore_barrier()

        def run_edges(tab):
            def chunk(c, carry):
                pltpu.async_copy(tab.at[srcv.at[c]], rows, sem).wait()
                pltpu.sync_copy(rows, acc.at[dstv.at[c]], add=True)
                if with_cnt:
                    @pl.when(cid == 0)
                    def _():
                        pltpu.sync_copy(onesv, cacc.at[dstv.at[c]], add=True)
                return carry

            lax.fori_loop(0, NCHUNK, chunk, 0)

        @pl.when(cid == 0)
        def _():
            run_edges(xa_h)

        @pl.when(cid == 1)
        def _():
            run_edges(xb_h)

        plsc.subcore_barrier()

        @pl.when(cid == 0)
        def _():
            pltpu.sync_copy(acc.at[rslice], outa_h.at[rslice])
            if with_cnt:
                pltpu.sync_copy(cacc.at[rslice], cnt_h.at[rslice])

        @pl.when(cid == 1)
        def _():
            pltpu.sync_copy(acc.at[rslice], outb_h.at[rslice])

    return k


_sc_scatter_cnt = _make_sc_scatter(True)
_sc_scatter = _make_sc_scatter(False)


def _tc_layer(sa, sb, cnt, pa, pb, Wl, b, Wr, relu, half_out):
    R = 1024
    grid = NP // R

    def body(sa_r, sb_r, c_r, pa_r, pb_r, wlt_r, wlb_r, b_r, wrt_r, wrb_r,
             *outs):
        inv = 1.0 / jnp.maximum(c_r[...], 1.0)
        o = jnp.dot(sa_r[...] * inv, wlt_r[...],
                    preferred_element_type=jnp.float32)
        o += jnp.dot(sb_r[...] * inv, wlb_r[...],
                     preferred_element_type=jnp.float32)
        o += jnp.dot(pa_r[...], wrt_r[...], preferred_element_type=jnp.float32)
        o += jnp.dot(pb_r[...], wrb_r[...], preferred_element_type=jnp.float32)
        o += b_r[...]
        if relu:
            o = jnp.maximum(o, 0.0)
        if half_out:
            outs[0][...] = o[:, :DH]
            outs[1][...] = o[:, DH:]
        else:
            outs[0][...] = o

    row_spec = lambda w: pl.BlockSpec((R, w), lambda i: (i, 0))
    full_spec = lambda s: pl.BlockSpec(s, lambda i: (0, 0))
    in_specs = [row_spec(DH), row_spec(DH), row_spec(1), row_spec(DH),
                row_spec(DH), full_spec((DH, D)), full_spec((DH, D)),
                full_spec((1, D)), full_spec((DH, D)), full_spec((DH, D))]
    if half_out:
        out_specs = [row_spec(DH), row_spec(DH)]
        out_shape = [jax.ShapeDtypeStruct((NP, DH), jnp.float32)] * 2
    else:
        out_specs = [row_spec(D)]
        out_shape = [jax.ShapeDtypeStruct((NP, D), jnp.float32)]

    return pl.pallas_call(body, grid=(grid,), in_specs=in_specs,
                          out_specs=out_specs, out_shape=out_shape)(
        sa, sb, cnt, pa, pb, Wl[:DH], Wl[DH:], b.reshape(1, D), Wr[:DH],
        Wr[DH:])


def kernel(x_SRC, edge_index, W_l1, b1, W_r1, W_l2, b2, W_r2):
    xp = jnp.pad(x_SRC, ((0, NP - N), (0, 0)))
    xa = xp[:, :DH]
    xb = xp[:, DH:]
    src3 = jnp.concatenate(
        [edge_index[0], jnp.zeros((EP - E,), jnp.int32)]).reshape(
            16, NCHUNK, CHUNK)
    dst3 = jnp.concatenate(
        [edge_index[1], jnp.full((EP - E,), N, jnp.int32)]).reshape(
            16, NCHUNK, CHUNK)
    zr = jnp.zeros((RPT, DH), jnp.float32)
    zc = jnp.zeros((RPT,), jnp.float32)
    ones = jnp.ones((CHUNK,), jnp.float32)

    sa1, sb1, cntp = _sc_scatter_cnt(xa, xb, src3, dst3, zr, zc, ones)
    cnt = cntp.reshape(NP, 1)

    ha, hb = _tc_layer(sa1, sb1, cnt, xa, xb, W_l1, b1, W_r1,
                       relu=True, half_out=True)

    sa2, sb2, _ = _sc_scatter(ha, hb, src3, dst3, zr, zc, ones)

    (out,) = _tc_layer(sa2, sb2, cnt, ha, hb, W_l2, b2, W_r2,
                       relu=False, half_out=False)
    return out[:N]


# speedup vs baseline: 1.3413x; 1.3413x over previous
import functools

import jax
import jax.numpy as jnp
from jax import lax
from jax.experimental import pallas as pl
from jax.experimental.pallas import tpu as pltpu
from jax.experimental.pallas import tpu_sc as plsc

N = 10000
NP = 10240
RPT = NP // 16
D = 256
DH = 128
E = 160000
EPT = E // 16
CHUNK = 125
NCHUNK = EPT // CHUNK


def _make_sc_scatter(with_cnt: bool):
    mesh = plsc.VectorSubcoreMesh(core_axis_name="c", subcore_axis_name="s")
    out_type = [
        jax.ShapeDtypeStruct((NP, DH), jnp.float32),
        jax.ShapeDtypeStruct((NP, DH), jnp.float32),
        jax.ShapeDtypeStruct((NP,), jnp.float32),
    ]
    scratch_types = [
        pltpu.VMEM((NCHUNK, CHUNK), jnp.int32),
        pltpu.VMEM((NCHUNK, CHUNK), jnp.int32),
        pltpu.VMEM((CHUNK, DH), jnp.float32),
        pltpu.VMEM((CHUNK,), jnp.float32),
        pltpu.VMEM_SHARED((NP, DH), jnp.float32),
        pltpu.VMEM_SHARED((NP,), jnp.float32),
        pltpu.SemaphoreType.DMA,
        pltpu.SemaphoreType.DMA,
    ]

    @functools.partial(pl.kernel, mesh=mesh, out_type=out_type,
                       scratch_types=scratch_types)
    def k(xa_h, xb_h, src_h, dst_h, zr_h, zc_h, ones_h,
          outa_h, outb_h, cnt_h,
          srcv, dstv, rows, onesv, acc, cacc, sem, isem):
        cid = lax.axis_index("c")
        sid = lax.axis_index("s")
        rslice = pl.ds(sid * RPT, RPT)

        pltpu.sync_copy(src_h.at[sid], srcv)
        pltpu.sync_copy(dst_h.at[sid], dstv)
        pltpu.sync_copy(zr_h, acc.at[rslice])
        if with_cnt:
            @pl.when(cid == 0)
            def _():
                pltpu.sync_copy(ones_h, onesv)
                pltpu.sync_copy(zc_h, cacc.at[rslice])
        plsc.subcore_barrier()

        def run_edges(tab):
            def chunk(c, carry):
                pltpu.async_copy(tab.at[srcv.at[c]], rows, sem).wait()
                pltpu.sync_copy(rows, acc.at[dstv.at[c]], add=True)
                if with_cnt:
                    @pl.when(cid == 0)
                    def _():
                        pltpu.sync_copy(onesv, cacc.at[dstv.at[c]], add=True)
                return carry

            lax.fori_loop(0, NCHUNK, chunk, 0)

        @pl.when(cid == 0)
        def _():
            run_edges(xa_h)

        @pl.when(cid == 1)
        def _():
            run_edges(xb_h)

        plsc.subcore_barrier()

        @pl.when(cid == 0)
        def _():
            pltpu.sync_copy(acc.at[rslice], outa_h.at[rslice])
            if with_cnt:
                pltpu.sync_copy(cacc.at[rslice], cnt_h.at[rslice])

        @pl.when(cid == 1)
        def _():
            pltpu.sync_copy(acc.at[rslice], outb_h.at[rslice])

    return k


_sc_scatter_cnt = _make_sc_scatter(True)
_sc_scatter = _make_sc_scatter(False)


def _tc_layer(sa, sb, cnt, pa, pb, Wl, b, Wr, relu, half_out):
    R = 1024
    grid = NP // R

    def body(sa_r, sb_r, c_r, pa_r, pb_r, wlt_r, wlb_r, b_r, wrt_r, wrb_r,
             *outs):
        inv = 1.0 / jnp.maximum(c_r[...], 1.0)
        o = jnp.dot(sa_r[...] * inv, wlt_r[...],
                    preferred_element_type=jnp.float32)
        o += jnp.dot(sb_r[...] * inv, wlb_r[...],
                     preferred_element_type=jnp.float32)
        o += jnp.dot(pa_r[...], wrt_r[...], preferred_element_type=jnp.float32)
        o += jnp.dot(pb_r[...], wrb_r[...], preferred_element_type=jnp.float32)
        o += b_r[...]
        if relu:
            o = jnp.maximum(o, 0.0)
        if half_out:
            outs[0][...] = o[:, :DH]
            outs[1][...] = o[:, DH:]
        else:
            outs[0][...] = o

    row_spec = lambda w: pl.BlockSpec((R, w), lambda i: (i, 0))
    full_spec = lambda s: pl.BlockSpec(s, lambda i: (0, 0))
    in_specs = [row_spec(DH), row_spec(DH), row_spec(1), row_spec(DH),
                row_spec(DH), full_spec((DH, D)), full_spec((DH, D)),
                full_spec((1, D)), full_spec((DH, D)), full_spec((DH, D))]
    if half_out:
        out_specs = [row_spec(DH), row_spec(DH)]
        out_shape = [jax.ShapeDtypeStruct((NP, DH), jnp.float32)] * 2
    else:
        out_specs = [row_spec(D)]
        out_shape = [jax.ShapeDtypeStruct((NP, D), jnp.float32)]

    return pl.pallas_call(body, grid=(grid,), in_specs=in_specs,
                          out_specs=out_specs, out_shape=out_shape)(
        sa, sb, cnt, pa, pb, Wl[:DH], Wl[DH:], b.reshape(1, D), Wr[:DH],
        Wr[DH:])


def kernel(x_SRC, edge_index, W_l1, b1, W_r1, W_l2, b2, W_r2):
    xp = jnp.pad(x_SRC, ((0, NP - N), (0, 0)))
    xa = xp[:, :DH]
    xb = xp[:, DH:]
    src3 = edge_index[0].reshape(16, NCHUNK, CHUNK)
    dst3 = edge_index[1].reshape(16, NCHUNK, CHUNK)
    zr = jnp.zeros((RPT, DH), jnp.float32)
    zc = jnp.zeros((RPT,), jnp.float32)
    ones = jnp.ones((CHUNK,), jnp.float32)

    sa1, sb1, cntp = _sc_scatter_cnt(xa, xb, src3, dst3, zr, zc, ones)
    cnt = cntp.reshape(NP, 1)

    ha, hb = _tc_layer(sa1, sb1, cnt, xa, xb, W_l1, b1, W_r1,
                       relu=True, half_out=True)

    sa2, sb2, _ = _sc_scatter(ha, hb, src3, dst3, zr, zc, ones)

    (out,) = _tc_layer(sa2, sb2, cnt, ha, hb, W_l2, b2, W_r2,
                       relu=False, half_out=False)
    return out[:N]
